# baseline (speedup 1.0000x reference)
import jax
import jax.numpy as jnp
from jax import lax
from jax.experimental import pallas as pl
from jax.experimental.pallas import tpu as pltpu
from jax.experimental.pallas import tpu_sc as plsc

NC = 2
NS = 16
L = 16
NW = NC * NS

D = 128
HD = D // 2
CHUNK = 32
NBUF = 8
UNROLL = 4


def _sc_body(h_hbm, r_hbm, t_hbm, ent_hbm, rel_hbm, out_hbm,
             hidx, ridx, tidx, bufrows, partials, outv, sem_idx,
             sem0, sem1, sem2, sem3, sem4, sem5, sem6, sem7):
    sems = (sem0, sem1, sem2, sem3, sem4, sem5, sem6, sem7)
    rows_per_w = out_hbm.shape[0] // NW
    n_chunks = rows_per_w // CHUNK
    wid = lax.axis_index("s") * NC + lax.axis_index("c")
    base = wid * rows_per_w

    for src, dst in ((h_hbm, hidx), (r_hbm, ridx), (t_hbm, tidx)):
        pltpu.async_copy(src.at[pl.ds(base, rows_per_w)], dst, sem_idx)
    for dst in (hidx, ridx, tidx):
        pltpu.make_async_copy(h_hbm.at[pl.ds(0, rows_per_w)], dst, sem_idx).wait()

    def chunk_srcs(ci):
        sl = pl.ds(ci * CHUNK, CHUNK)
        return ((ent_hbm, hidx.at[sl]), (rel_hbm, ridx.at[sl]),
                (ent_hbm, tidx.at[sl]))

    def buf_ref(s, k):
        return bufrows.at[pl.ds((s * 3 + k) * CHUNK, CHUNK), :]

    def fire(ci, s):
        for k, (table, idx) in enumerate(chunk_srcs(ci)):
            pltpu.async_copy(table.at[idx], buf_ref(s, k), sems[s])

    def wait_bufs(ci, s):
        for k, (table, idx) in enumerate(chunk_srcs(ci)):
            pltpu.make_async_copy(table.at[idx], buf_ref(s, k),
                                  sems[s]).wait()

    def compute_chunk(ci, s):
        hb = buf_ref(s, 0)
        rb = buf_ref(s, 1)
        tb = buf_ref(s, 2)

        def row_score(b):
            acc = None
            for j in range(HD // L):
                hr = hb[b, pl.ds(j * L, L)]
                hi = hb[b, pl.ds(HD + j * L, L)]
                rr = rb[b, pl.ds(j * L, L)]
                ri = rb[b, pl.ds(HD + j * L, L)]
                tr = tb[b, pl.ds(j * L, L)]
                ti = tb[b, pl.ds(HD + j * L, L)]
                part = tr * (hr * rr - hi * ri) + ti * (hr * ri + hi * rr)
                acc = part if acc is None else acc + part
            return jnp.cumsum(acc)

        def group_body(g, carry):
            b0 = g * UNROLL
            for u in range(UNROLL):
                partials[b0 + u, :] = row_score(b0 + u)
            return carry

        lax.fori_loop(0, CHUNK // UNROLL, group_body, 0)

        last = jnp.full((L,), L - 1, jnp.int32)

        def extract_body(g, carry):
            rowv = g * L + lax.iota(jnp.int32, L)
            outv[pl.ds(ci * CHUNK + g * L, L)] = plsc.load_gather(
                partials, [rowv, last])
            return carry

        lax.fori_loop(0, CHUNK // L, extract_body, 0)

    for ci in range(NBUF - 1):
        fire(ci, ci)

    def ring_body(i, carry):
        for s in range(NBUF):
            ci = i * NBUF + s
            wait_bufs(ci, s)

            @pl.when(ci + (NBUF - 1) < n_chunks)
            def _prefetch():
                fire(ci + (NBUF - 1), (s + NBUF - 1) % NBUF)

            compute_chunk(ci, s)
        return carry

    lax.fori_loop(0, n_chunks // NBUF, ring_body, 0)
    pltpu.sync_copy(outv, out_hbm.at[pl.ds(base, rows_per_w)])


def kernel(h, r, t, ent_table, rel_table):
    B = h.shape[0]
    rows_per_w = B // NW
    mesh = plsc.VectorSubcoreMesh(
        core_axis_name="c", subcore_axis_name="s",
        num_cores=NC, num_subcores=NS)
    run = pl.kernel(
        _sc_body,
        out_type=jax.ShapeDtypeStruct((B,), jnp.float32),
        mesh=mesh,
        scratch_types=[
            pltpu.VMEM((rows_per_w,), jnp.int32),
            pltpu.VMEM((rows_per_w,), jnp.int32),
            pltpu.VMEM((rows_per_w,), jnp.int32),
            pltpu.VMEM((NBUF * 3 * CHUNK, D), jnp.float32),
            pltpu.VMEM((CHUNK, L), jnp.float32),
            pltpu.VMEM((rows_per_w,), jnp.float32),
            pltpu.SemaphoreType.DMA,
            pltpu.SemaphoreType.DMA,
            pltpu.SemaphoreType.DMA,
            pltpu.SemaphoreType.DMA,
            pltpu.SemaphoreType.DMA,
            pltpu.SemaphoreType.DMA,
            pltpu.SemaphoreType.DMA,
            pltpu.SemaphoreType.DMA,
            pltpu.SemaphoreType.DMA,
        ],
        compiler_params=pltpu.CompilerParams(
            needs_layout_passes=False,
            disable_bounds_checks=True,
            disable_semaphore_checks=True,
            skip_device_barrier=True,
        ),
    )
    return run(h, r, t, ent_table, rel_table)[:, None]


# speedup vs baseline: 1.0147x; 1.0147x over previous
import jax
import jax.numpy as jnp
from jax import lax
from jax.experimental import pallas as pl
from jax.experimental.pallas import tpu as pltpu
from jax.experimental.pallas import tpu_sc as plsc

NC = 2
NS = 16
L = 16
NW = NC * NS

D = 128
HD = D // 2
CHUNK = 64
NBUF = 4
UNROLL = 4


def _sc_body(h_hbm, r_hbm, t_hbm, ent_hbm, rel_hbm, out_hbm,
             hidx, ridx, tidx, bufrows, partials, outv, sem_idx,
             sems):
    rows_per_w = out_hbm.shape[0] // NW
    n_chunks = rows_per_w // CHUNK
    wid = lax.axis_index("s") * NC + lax.axis_index("c")
    base = wid * rows_per_w

    for src, dst in ((h_hbm, hidx), (r_hbm, ridx), (t_hbm, tidx)):
        pltpu.async_copy(src.at[pl.ds(base, rows_per_w)], dst, sem_idx)
    for dst in (hidx, ridx, tidx):
        pltpu.make_async_copy(h_hbm.at[pl.ds(0, rows_per_w)], dst, sem_idx).wait()

    def chunk_srcs(ci):
        sl = pl.ds(ci * CHUNK, CHUNK)
        return ((ent_hbm, hidx.at[sl]), (rel_hbm, ridx.at[sl]),
                (ent_hbm, tidx.at[sl]))

    def buf_ref(s, k):
        return bufrows.at[pl.ds((s * 3 + k) * CHUNK, CHUNK), :]

    def fire(ci, s):
        for k, (table, idx) in enumerate(chunk_srcs(ci)):
            pltpu.async_copy(table.at[idx], buf_ref(s, k), sems.at[s])

    def wait_bufs(ci, s):
        for k, (table, idx) in enumerate(chunk_srcs(ci)):
            pltpu.make_async_copy(table.at[idx], buf_ref(s, k),
                                  sems.at[s]).wait()

    def compute_chunk(ci, s):
        hb = buf_ref(s, 0)
        rb = buf_ref(s, 1)
        tb = buf_ref(s, 2)

        def row_score(b):
            acc = None
            for j in range(HD // L):
                hr = hb[b, pl.ds(j * L, L)]
                hi = hb[b, pl.ds(HD + j * L, L)]
                rr = rb[b, pl.ds(j * L, L)]
                ri = rb[b, pl.ds(HD + j * L, L)]
                tr = tb[b, pl.ds(j * L, L)]
                ti = tb[b, pl.ds(HD + j * L, L)]
                part = tr * (hr * rr - hi * ri) + ti * (hr * ri + hi * rr)
                acc = part if acc is None else acc + part
            return jnp.cumsum(acc)

        def group_body(g, carry):
            b0 = g * UNROLL
            for u in range(UNROLL):
                partials[b0 + u, :] = row_score(b0 + u)
            return carry

        lax.fori_loop(0, CHUNK // UNROLL, group_body, 0)

        last = jnp.full((L,), L - 1, jnp.int32)

        def extract_body(g, carry):
            rowv = g * L + lax.iota(jnp.int32, L)
            outv[pl.ds(ci * CHUNK + g * L, L)] = plsc.load_gather(
                partials, [rowv, last])
            return carry

        lax.fori_loop(0, CHUNK // L, extract_body, 0)

    for ci in range(NBUF - 1):
        fire(ci, ci)

    def ring_body(ci, carry):
        s = lax.rem(ci, NBUF)
        wait_bufs(ci, s)

        @pl.when(ci + (NBUF - 1) < n_chunks)
        def _prefetch():
            fire(ci + (NBUF - 1), lax.rem(ci + (NBUF - 1), NBUF))

        compute_chunk(ci, s)
        return carry

    lax.fori_loop(0, n_chunks, ring_body, 0)
    pltpu.sync_copy(outv, out_hbm.at[pl.ds(base, rows_per_w)])


def kernel(h, r, t, ent_table, rel_table):
    B = h.shape[0]
    rows_per_w = B // NW
    mesh = plsc.VectorSubcoreMesh(
        core_axis_name="c", subcore_axis_name="s",
        num_cores=NC, num_subcores=NS)
    run = pl.kernel(
        _sc_body,
        out_type=jax.ShapeDtypeStruct((B,), jnp.float32),
        mesh=mesh,
        scratch_types=[
            pltpu.VMEM((rows_per_w,), jnp.int32),
            pltpu.VMEM((rows_per_w,), jnp.int32),
            pltpu.VMEM((rows_per_w,), jnp.int32),
            pltpu.VMEM((NBUF * 3 * CHUNK, D), jnp.float32),
            pltpu.VMEM((CHUNK, L), jnp.float32),
            pltpu.VMEM((rows_per_w,), jnp.float32),
            pltpu.SemaphoreType.DMA,
            pltpu.SemaphoreType.DMA((NBUF,)),
        ],
        compiler_params=pltpu.CompilerParams(
            needs_layout_passes=False,
            disable_bounds_checks=True,
            disable_semaphore_checks=True,
            skip_device_barrier=True,
        ),
    )
    return run(h, r, t, ent_table, rel_table)[:, None]


# speedup vs baseline: 1.1262x; 1.1098x over previous
import jax
import jax.numpy as jnp
from jax import lax
from jax.experimental import pallas as pl
from jax.experimental.pallas import tpu as pltpu
from jax.experimental.pallas import tpu_sc as plsc

NC = 2
NS = 16
L = 16
NW = NC * NS

D = 128
HD = D // 2
CHUNK = 64
NBUF = 4
UNROLL = 4


def _sc_body(h_hbm, r_hbm, t_hbm, ent_hbm, rel_hbm, out_hbm,
             hidx, ridx, tidx, bufrows, partials, outv, sem_idx,
             sem0, sem1, sem2, sem3):
    sems = (sem0, sem1, sem2, sem3)
    rows_per_w = out_hbm.shape[0] // NW
    n_chunks = rows_per_w // CHUNK
    wid = lax.axis_index("s") * NC + lax.axis_index("c")
    base = wid * rows_per_w

    for src, dst in ((h_hbm, hidx), (r_hbm, ridx), (t_hbm, tidx)):
        pltpu.async_copy(src.at[pl.ds(base, rows_per_w)], dst, sem_idx)
    for dst in (hidx, ridx, tidx):
        pltpu.make_async_copy(h_hbm.at[pl.ds(0, rows_per_w)], dst, sem_idx).wait()

    def chunk_srcs(ci):
        sl = pl.ds(ci * CHUNK, CHUNK)
        return ((ent_hbm, hidx.at[sl]), (rel_hbm, ridx.at[sl]),
                (ent_hbm, tidx.at[sl]))

    def buf_ref(s, k):
        return bufrows.at[pl.ds((s * 3 + k) * CHUNK, CHUNK), :]

    def fire(ci, s):
        for k, (table, idx) in enumerate(chunk_srcs(ci)):
            pltpu.async_copy(table.at[idx], buf_ref(s, k), sems[s])

    def wait_bufs(ci, s):
        for k, (table, idx) in enumerate(chunk_srcs(ci)):
            pltpu.make_async_copy(table.at[idx], buf_ref(s, k),
                                  sems[s]).wait()

    def compute_chunk(ci, s):
        hb = buf_ref(s, 0)
        rb = buf_ref(s, 1)
        tb = buf_ref(s, 2)

        def row_score(b):
            acc = None
            for j in range(HD // L):
                hr = hb[b, pl.ds(j * L, L)]
                hi = hb[b, pl.ds(HD + j * L, L)]
                rr = rb[b, pl.ds(j * L, L)]
                ri = rb[b, pl.ds(HD + j * L, L)]
                tr = tb[b, pl.ds(j * L, L)]
                ti = tb[b, pl.ds(HD + j * L, L)]
                part = tr * (hr * rr - hi * ri) + ti * (hr * ri + hi * rr)
                acc = part if acc is None else acc + part
            return jnp.cumsum(acc)

        def group_body(g, carry):
            b0 = g * UNROLL
            for u in range(UNROLL):
                partials[b0 + u, :] = row_score(b0 + u)
            return carry

        lax.fori_loop(0, CHUNK // UNROLL, group_body, 0)

        last = jnp.full((L,), L - 1, jnp.int32)

        def extract_body(g, carry):
            rowv = g * L + lax.iota(jnp.int32, L)
            outv[pl.ds(ci * CHUNK + g * L, L)] = plsc.load_gather(
                partials, [rowv, last])
            return carry

        lax.fori_loop(0, CHUNK // L, extract_body, 0)

    for ci in range(NBUF - 1):
        fire(ci, ci)

    def ring_body(i, carry):
        for s in range(NBUF):
            ci = i * NBUF + s
            wait_bufs(ci, s)

            @pl.when(ci + (NBUF - 1) < n_chunks)
            def _prefetch():
                fire(ci + (NBUF - 1), (s + NBUF - 1) % NBUF)

            compute_chunk(ci, s)
        return carry

    lax.fori_loop(0, n_chunks // NBUF, ring_body, 0)
    pltpu.sync_copy(outv, out_hbm.at[pl.ds(base, rows_per_w)])


def kernel(h, r, t, ent_table, rel_table):
    B = h.shape[0]
    rows_per_w = B // NW
    mesh = plsc.VectorSubcoreMesh(
        core_axis_name="c", subcore_axis_name="s",
        num_cores=NC, num_subcores=NS)
    run = pl.kernel(
        _sc_body,
        out_type=jax.ShapeDtypeStruct((B,), jnp.float32),
        mesh=mesh,
        scratch_types=[
            pltpu.VMEM((rows_per_w,), jnp.int32),
            pltpu.VMEM((rows_per_w,), jnp.int32),
            pltpu.VMEM((rows_per_w,), jnp.int32),
            pltpu.VMEM((NBUF * 3 * CHUNK, D), jnp.float32),
            pltpu.VMEM((CHUNK, L), jnp.float32),
            pltpu.VMEM((rows_per_w,), jnp.float32),
            pltpu.SemaphoreType.DMA,
            pltpu.SemaphoreType.DMA,
            pltpu.SemaphoreType.DMA,
            pltpu.SemaphoreType.DMA,
            pltpu.SemaphoreType.DMA,
        ],
        compiler_params=pltpu.CompilerParams(
            needs_layout_passes=False,
            disable_bounds_checks=True,
            disable_semaphore_checks=True,
            skip_device_barrier=True,
        ),
    )
    return run(h, r, t, ent_table, rel_table)[:, None]


# speedup vs baseline: 1.1278x; 1.0015x over previous
import jax
import jax.numpy as jnp
from jax import lax
from jax.experimental import pallas as pl
from jax.experimental.pallas import tpu as pltpu
from jax.experimental.pallas import tpu_sc as plsc

NC = 2
NS = 16
L = 16
NW = NC * NS

D = 128
HD = D // 2
CHUNK = 64
NBUF = 4
UNROLL = 4


def _sc_body(h_hbm, r_hbm, t_hbm, ent_hbm, rel_hbm, out_hbm,
             hidx, ridx, tidx, bufrows, partials, outv, sem_idx,
             sem0, sem1, sem2, sem3):
    sems = (sem0, sem1, sem2, sem3)
    rows_per_w = out_hbm.shape[0] // NW
    n_chunks = rows_per_w // CHUNK
    wid = lax.axis_index("s") * NC + lax.axis_index("c")
    base = wid * rows_per_w

    for src, dst in ((h_hbm, hidx), (r_hbm, ridx), (t_hbm, tidx)):
        pltpu.async_copy(src.at[pl.ds(base, rows_per_w)], dst, sem_idx)
    for dst in (hidx, ridx, tidx):
        pltpu.make_async_copy(h_hbm.at[pl.ds(0, rows_per_w)], dst, sem_idx).wait()

    def chunk_srcs(ci):
        sl = pl.ds(ci * CHUNK, CHUNK)
        return ((ent_hbm, hidx.at[sl]), (rel_hbm, ridx.at[sl]),
                (ent_hbm, tidx.at[sl]))

    def buf_ref(s, k):
        return bufrows.at[pl.ds((s * 3 + k) * CHUNK, CHUNK), :]

    def fire(ci, s):
        for k, (table, idx) in enumerate(chunk_srcs(ci)):
            pltpu.async_copy(table.at[idx], buf_ref(s, k), sems[s])

    def wait_bufs(ci, s):
        for k, (table, idx) in enumerate(chunk_srcs(ci)):
            pltpu.make_async_copy(table.at[idx], buf_ref(s, k),
                                  sems[s]).wait()

    def compute_chunk(ci, s):
        hb = buf_ref(s, 0)
        rb = buf_ref(s, 1)
        tb = buf_ref(s, 2)

        def row_score(b):
            acc = None
            for j in range(HD // L):
                hr = hb[b, pl.ds(j * L, L)]
                hi = hb[b, pl.ds(HD + j * L, L)]
                rr = rb[b, pl.ds(j * L, L)]
                ri = rb[b, pl.ds(HD + j * L, L)]
                tr = tb[b, pl.ds(j * L, L)]
                ti = tb[b, pl.ds(HD + j * L, L)]
                part = tr * (hr * rr - hi * ri) + ti * (hr * ri + hi * rr)
                acc = part if acc is None else acc + part
            return jnp.cumsum(acc)

        def group_body(g, carry):
            b0 = g * UNROLL
            for u in range(UNROLL):
                partials[b0 + u, :] = row_score(b0 + u)
            return carry

        lax.fori_loop(0, CHUNK // UNROLL, group_body, 0)

        last = jnp.full((L,), L - 1, jnp.int32)

        def extract_body(g, carry):
            rowv = g * L + lax.iota(jnp.int32, L)
            outv[pl.ds(ci * CHUNK + g * L, L)] = plsc.load_gather(
                partials, [rowv, last])
            return carry

        lax.fori_loop(0, CHUNK // L, extract_body, 0)

    for ci in range(NBUF - 1):
        fire(ci, ci)

    def ring_body(i, carry):
        for s in range(NBUF):
            ci = i * NBUF + s
            wait_bufs(ci, s)

            @pl.when(ci + (NBUF - 1) < n_chunks)
            def _prefetch():
                fire(ci + (NBUF - 1), (s + NBUF - 1) % NBUF)

            compute_chunk(ci, s)
        return carry

    lax.fori_loop(0, n_chunks // NBUF, ring_body, 0)
    pltpu.sync_copy(outv, out_hbm.at[pl.ds(base, rows_per_w)])


def kernel(h, r, t, ent_table, rel_table):
    B = h.shape[0]
    rows_per_w = B // NW
    mesh = plsc.VectorSubcoreMesh(
        core_axis_name="c", subcore_axis_name="s",
        num_cores=NC, num_subcores=NS)
    run = pl.kernel(
        _sc_body,
        out_type=jax.ShapeDtypeStruct((B,), jnp.float32),
        mesh=mesh,
        scratch_types=[
            pltpu.VMEM((rows_per_w,), jnp.int32),
            pltpu.VMEM((rows_per_w,), jnp.int32),
            pltpu.VMEM((rows_per_w,), jnp.int32),
            pltpu.VMEM((NBUF * 3 * CHUNK, D), jnp.float32),
            pltpu.VMEM((CHUNK, L), jnp.float32),
            pltpu.VMEM((rows_per_w,), jnp.float32),
            pltpu.SemaphoreType.DMA,
            pltpu.SemaphoreType.DMA,
            pltpu.SemaphoreType.DMA,
            pltpu.SemaphoreType.DMA,
            pltpu.SemaphoreType.DMA,
        ],
        compiler_params=pltpu.CompilerParams(needs_layout_passes=False),
    )
    return run(h, r, t, ent_table, rel_table)[:, None]
